# Initial kernel scaffold; baseline (speedup 1.0000x reference)
import jax
import jax.numpy as jnp
from jax.experimental import pallas as pl


def kernel(feat, edge_index, W1, b1, W2, b2):
    raise NotImplementedError("write your pallas kernel here")


# speedup vs baseline: 4.8676x; 4.8676x over previous
import functools

import jax
import jax.numpy as jnp
from jax import lax
from jax.experimental import pallas as pl
from jax.experimental.pallas import tpu as pltpu
from jax.experimental.pallas import tpu_sc as plsc

N = 10000
E = 320000
D0 = 128
D1 = 256

NC = 2
NS = 16
NW = NC * NS
LANE = 128
NB = -(-E // (NW * LANE))
E_PAD = NW * NB * LANE
N_PAD = 10240
ROWS_PT = N_PAD // NS
BLK = 1000

_mesh = plsc.VectorSubcoreMesh(core_axis_name="c", subcore_axis_name="s")


NB2 = -(-E // (NS * LANE))
E_PAD2 = NS * LANE * NB2


@functools.partial(
    pl.kernel,
    out_type=jax.ShapeDtypeStruct((NC, N_PAD, D0), jnp.float32),
    mesh=_mesh,
    scratch_types=[
        pltpu.VMEM((NB2, LANE), jnp.int32),
        pltpu.VMEM((LANE, D0), jnp.float32),
        pltpu.VMEM((64, D0), jnp.float32),
        pltpu.VMEM_SHARED((N_PAD, D0), jnp.float32),
    ],
)
def _deg_kernel(edges_h, deg_h, idx_v, ones_v, z_v, deg_sp):
    c = lax.axis_index("c")
    s = lax.axis_index("s")
    base = s * ROWS_PT

    def fill(i, _):
        for k in range(D0 // 16):
            ones_v[i, pl.ds(k * 16, 16)] = jnp.ones((16,), jnp.float32)
        return 0

    lax.fori_loop(0, LANE, fill, 0)

    def fill_z(i, _):
        for k in range(D0 // 16):
            z_v[i, pl.ds(k * 16, 16)] = jnp.zeros((16,), jnp.float32)
        return 0

    lax.fori_loop(0, 64, fill_z, 0)
    for t in range(ROWS_PT // 64):
        pltpu.sync_copy(z_v, deg_sp.at[pl.ds(base + t * 64, 64)])
    pltpu.sync_copy(edges_h.at[c, s], idx_v)
    plsc.subcore_barrier()

    def body(j, _):
        pltpu.sync_copy(ones_v, deg_sp.at[idx_v.at[j]], add=True)
        return 0

    lax.fori_loop(0, NB2, body, 0)
    plsc.subcore_barrier()
    pltpu.sync_copy(deg_sp.at[pl.ds(base, ROWS_PT)],
                    deg_h.at[c, pl.ds(base, ROWS_PT)])


@functools.partial(
    pl.kernel,
    out_type=jax.ShapeDtypeStruct((NC, N_PAD, D0), jnp.float32),
    mesh=_mesh,
    scratch_types=[
        pltpu.VMEM((NB, LANE), jnp.int32),
        pltpu.VMEM((NB, LANE), jnp.int32),
        pltpu.VMEM((LANE, D0), jnp.float32),
        pltpu.VMEM((64, D0), jnp.float32),
        pltpu.VMEM_SHARED((N_PAD, D0), jnp.float32),
        pltpu.SemaphoreType.DMA,
    ],
)
def _spmm_kernel(x_h, src_h, dst_h, out_h,
                 src_v, dst_v, rows_v, z_v, acc_sp, sem):
    c = lax.axis_index("c")
    s = lax.axis_index("s")
    wid = c * NS + s
    base = s * ROWS_PT

    def fill(i, _):
        for k in range(D0 // 16):
            z_v[i, pl.ds(k * 16, 16)] = jnp.zeros((16,), jnp.float32)
        return 0

    lax.fori_loop(0, 64, fill, 0)
    for t in range(ROWS_PT // 64):
        pltpu.sync_copy(z_v, acc_sp.at[pl.ds(base + t * 64, 64)])
    pltpu.sync_copy(src_h.at[wid], src_v)
    pltpu.sync_copy(dst_h.at[wid], dst_v)
    plsc.subcore_barrier()

    def body(j, _):
        pltpu.async_copy(x_h.at[src_v.at[j]], rows_v, sem).wait()
        pltpu.sync_copy(rows_v, acc_sp.at[dst_v.at[j]], add=True)
        return 0

    lax.fori_loop(0, NB, body, 0)
    plsc.subcore_barrier()
    pltpu.sync_copy(acc_sp.at[pl.ds(base, ROWS_PT)],
                    out_h.at[c, pl.ds(base, ROWS_PT)])


def _norm(deg_block, plane):
    return lax.rsqrt(jnp.maximum(deg_block[plane, :, 0:1], 1.0))


def _scale_body(feat_ref, deg_ref, o_ref):
    o_ref[...] = feat_ref[...] * _norm(deg_ref, 0)


def _mid_body(p_ref, deg_ref, w1_ref, b1_ref, w2_ref, o_ref):
    agg = (p_ref[0] + p_ref[1]) * _norm(deg_ref, 1)
    y = jnp.maximum(
        jnp.dot(agg, w1_ref[...], preferred_element_type=jnp.float32)
        + b1_ref[...], 0.0)
    o_ref[...] = jnp.dot(y * _norm(deg_ref, 0), w2_ref[...],
                         preferred_element_type=jnp.float32)


def _final_body(q_ref, deg_ref, b2_ref, o_ref):
    o_ref[...] = jnp.maximum(
        (q_ref[0] + q_ref[1]) * _norm(deg_ref, 1) + b2_ref[...], 0.0)


_deg_spec = pl.BlockSpec((NC, BLK, D0), lambda i: (0, i, 0))
_part_spec = pl.BlockSpec((NC, BLK, D0), lambda i: (0, i, 0))
_row_spec = pl.BlockSpec((BLK, D0), lambda i: (i, 0))

_scale_call = pl.pallas_call(
    _scale_body,
    grid=(N // BLK,),
    in_specs=[_row_spec, _deg_spec],
    out_specs=_row_spec,
    out_shape=jax.ShapeDtypeStruct((N, D0), jnp.float32),
)

_mid_call = pl.pallas_call(
    _mid_body,
    grid=(N // BLK,),
    in_specs=[
        _part_spec, _deg_spec,
        pl.BlockSpec((D0, D1), lambda i: (0, 0)),
        pl.BlockSpec((1, D1), lambda i: (0, 0)),
        pl.BlockSpec((D1, D0), lambda i: (0, 0)),
    ],
    out_specs=_row_spec,
    out_shape=jax.ShapeDtypeStruct((N, D0), jnp.float32),
)

_final_call = pl.pallas_call(
    _final_body,
    grid=(N // BLK,),
    in_specs=[_part_spec, _deg_spec, pl.BlockSpec((1, D0), lambda i: (0, 0))],
    out_specs=_row_spec,
    out_shape=jax.ShapeDtypeStruct((N, D0), jnp.float32),
)


def kernel(feat, edge_index, W1, b1, W2, b2):
    src = edge_index[0]
    dst = edge_index[1]
    pad = E_PAD - E
    src_g = jnp.concatenate(
        [src, jnp.zeros((pad,), jnp.int32)]).reshape(NW, NB, LANE)
    dst_p = jnp.concatenate(
        [dst, jnp.full((pad,), N, jnp.int32)]).reshape(NW, NB, LANE)
    pad2 = E_PAD2 - E
    edges_d = jnp.stack([
        jnp.concatenate([src, jnp.full((pad2,), N, jnp.int32)]),
        jnp.concatenate([dst, jnp.full((pad2,), N, jnp.int32)]),
    ]).reshape(2, NS, NB2, LANE)

    deg = _deg_kernel(edges_d)

    xs = _scale_call(feat, deg)
    p = _spmm_kernel(xs, src_g, dst_p)
    z = _mid_call(p, deg, W1, b1.reshape(1, D1), W2)
    q = _spmm_kernel(z, src_g, dst_p)
    return _final_call(q, deg, b2.reshape(1, D0))
